# Initial kernel scaffold; baseline (speedup 1.0000x reference)
import jax
import jax.numpy as jnp
from jax.experimental import pallas as pl


def kernel(x, edge_index, align_w0, align_b0, attend_w0, attend_b0, gru_wih0, gru_whh0, gru_bih0, gru_bhh0, align_w1, align_b1, attend_w1, attend_b1, gru_wih1, gru_whh1, gru_bih1, gru_bhh1):
    raise NotImplementedError("write your pallas kernel here")


# speedup vs baseline: 17.5616x; 17.5616x over previous
import functools

import jax
import jax.numpy as jnp
from jax import lax
from jax.experimental import pallas as pl
from jax.experimental.pallas import tpu as pltpu
from jax.experimental.pallas import tpu_sc as plsc

N = 10000
E = 320000
D = 128

NPAD = 10240
NWORK = 32
W = NPAD // NWORK
LANES = 16
CHUNK = 10000


def _hist_body(dst_hbm, out_hbm, chunk_v, hist_v, out_v):
    c_idx = lax.axis_index("c")
    s_idx = lax.axis_index("s")
    wid = s_idx * 2 + c_idx
    base = wid * W
    lane_off = lax.iota(jnp.int32, LANES) * W
    ones = jnp.full((LANES,), 1.0, jnp.float32)
    zeros = jnp.zeros((LANES,), jnp.float32)

    def zero_body(i, carry):
        hist_v[pl.ds(i * LANES, LANES)] = zeros
        return carry

    lax.fori_loop(0, (LANES * W) // LANES, zero_body, 0)

    def chunk_body(ci, carry):
        pltpu.sync_copy(dst_hbm.at[pl.ds(ci * CHUNK, CHUNK)], chunk_v)

        def edge_body(i, inner):
            v = chunk_v[pl.ds(i * LANES, LANES)]
            local = v - base
            m = (local >= 0) & (local < W)
            lc = jnp.minimum(jnp.maximum(local, 0), W - 1)
            plsc.addupdate_scatter(hist_v, [lane_off + lc], ones, mask=m)
            return inner

        lax.fori_loop(0, CHUNK // LANES, edge_body, 0)
        return carry

    lax.fori_loop(0, E // CHUNK, chunk_body, 0)

    for c in range(W // LANES):
        acc = hist_v[pl.ds(c * LANES, LANES)]
        for l in range(1, LANES):
            acc = acc + hist_v[pl.ds(l * W + c * LANES, LANES)]
        out_v[pl.ds(c * LANES, LANES)] = acc
    pltpu.sync_copy(out_v, out_hbm.at[pl.ds(base, W)])


def _in_degree(dst):
    mesh = plsc.VectorSubcoreMesh(core_axis_name="c", subcore_axis_name="s")
    kern = functools.partial(
        pl.kernel,
        mesh=mesh,
        out_type=jax.ShapeDtypeStruct((NPAD,), jnp.float32),
        scratch_types=[
            pltpu.VMEM((CHUNK,), jnp.int32),
            pltpu.VMEM((LANES * W,), jnp.float32),
            pltpu.VMEM((W,), jnp.float32),
        ],
        compiler_params=pltpu.CompilerParams(needs_layout_passes=False),
    )(_hist_body)
    return kern(dst)


def _elu(x):
    return jnp.where(x > 0, x, jnp.exp(jnp.minimum(x, 0.0)) - 1.0)


def _dot_t(a, w):
    return lax.dot_general(a, w, (((1,), (1,)), ((), ())),
                           preferred_element_type=jnp.float32)


def _gru_layer(hs, tw, tb, wih, bih, whh, bhh):
    cs = _elu(_dot_t(hs, tw) + tb)
    gi = _dot_t(cs, wih) + bih
    gh = _dot_t(hs, whh) + bhh
    r = jax.nn.sigmoid(gi[:, :D] + gh[:, :D])
    z = jax.nn.sigmoid(gi[:, D:2 * D] + gh[:, D:2 * D])
    n = jnp.tanh(gi[:, 2 * D:] + r * gh[:, 2 * D:])
    return (1.0 - z) * n + z * hs


def _dense_body(cnt_ref, x_ref, tw0_ref, tb0_ref, wih0_ref, bih0_ref,
                whh0_ref, bhh0_ref, tw1_ref, tb1_ref, wih1_ref, bih1_ref,
                whh1_ref, bhh1_ref, out_ref):
    xv = x_ref[...]
    cnt = cnt_ref[...]
    s = jnp.sum(xv, axis=1, keepdims=True)
    hs0 = jnp.broadcast_to(s, xv.shape)
    h0 = _gru_layer(hs0, tw0_ref[...], tb0_ref[...], wih0_ref[...],
                    bih0_ref[...], whh0_ref[...], bhh0_ref[...])
    y = cnt * h0
    h1 = _gru_layer(y, tw1_ref[...], tb1_ref[...], wih1_ref[...],
                    bih1_ref[...], whh1_ref[...], bhh1_ref[...])
    out_ref[...] = cnt * h1


BLK = 1000


def _dense(cnt, x, tw0, tb0, wih0, bih0, whh0, bhh0,
           tw1, tb1, wih1, bih1, whh1, bhh1):
    grid = (N // BLK,)
    full = lambda shape: pl.BlockSpec(shape, lambda i: (0, 0))
    return pl.pallas_call(
        _dense_body,
        grid=grid,
        in_specs=[
            pl.BlockSpec((BLK, 1), lambda i: (i, 0)),
            pl.BlockSpec((BLK, D), lambda i: (i, 0)),
            full((D, D)), full((1, D)),
            full((3 * D, D)), full((1, 3 * D)),
            full((3 * D, D)), full((1, 3 * D)),
            full((D, D)), full((1, D)),
            full((3 * D, D)), full((1, 3 * D)),
            full((3 * D, D)), full((1, 3 * D)),
        ],
        out_specs=pl.BlockSpec((BLK, D), lambda i: (i, 0)),
        out_shape=jax.ShapeDtypeStruct((N, D), jnp.float32),
    )(cnt, x, tw0, tb0, wih0, bih0, whh0, bhh0,
      tw1, tb1, wih1, bih1, whh1, bhh1)


def kernel(x, edge_index, align_w0, align_b0, attend_w0, attend_b0,
           gru_wih0, gru_whh0, gru_bih0, gru_bhh0, align_w1, align_b1,
           attend_w1, attend_b1, gru_wih1, gru_whh1, gru_bih1, gru_bhh1):
    cnt = _in_degree(edge_index[1])[:N].reshape(N, 1)
    return _dense(
        cnt, x,
        attend_w0, attend_b0.reshape(1, D),
        gru_wih0, gru_bih0.reshape(1, 3 * D),
        gru_whh0, gru_bhh0.reshape(1, 3 * D),
        attend_w1, attend_b1.reshape(1, D),
        gru_wih1, gru_bih1.reshape(1, 3 * D),
        gru_whh1, gru_bhh1.reshape(1, 3 * D),
    )


# speedup vs baseline: 49.4269x; 2.8145x over previous
import functools

import jax
import jax.numpy as jnp
from jax import lax
from jax.experimental import pallas as pl
from jax.experimental.pallas import tpu as pltpu
from jax.experimental.pallas import tpu_sc as plsc

N = 10000
E = 320000
D = 128

NPAD = 10240
NWORK = 32
LANES = 16
NSUB = 4
EPW = E // NWORK
HSZ = NPAD * NSUB


def _hist_body(dst_hbm, out_hbm, chunk_v, hist_v):
    c_idx = lax.axis_index("c")
    s_idx = lax.axis_index("s")
    wid = s_idx * 2 + c_idx
    lane = lax.iota(jnp.int32, LANES)
    ones = jnp.full((LANES,), 1.0, jnp.float32)
    zeros = jnp.zeros((LANES,), jnp.float32)
    masks = [(lane >= 4 * g) & (lane < 4 * g + 4) for g in range(4)]

    pltpu.sync_copy(dst_hbm.at[pl.ds(wid * EPW, EPW)], chunk_v)

    def zero_body(i, carry):
        for j in range(8):
            hist_v[pl.ds((i * 8 + j) * LANES, LANES)] = zeros
        return carry

    lax.fori_loop(0, HSZ // (8 * LANES), zero_body, 0)

    def edge_body(i, carry):
        v = chunk_v[pl.ds(i * LANES, LANES)]
        t = v * NSUB + lane
        for g in range(4):
            plsc.addupdate_scatter(hist_v, [t - 4 * g], ones, mask=masks[g])
        return carry

    lax.fori_loop(0, EPW // LANES, edge_body, 0)

    pltpu.sync_copy(hist_v, out_hbm.at[wid])


def _in_degree(dst):
    mesh = plsc.VectorSubcoreMesh(core_axis_name="c", subcore_axis_name="s")
    kern = functools.partial(
        pl.kernel,
        mesh=mesh,
        out_type=jax.ShapeDtypeStruct((NWORK, HSZ), jnp.float32),
        scratch_types=[
            pltpu.VMEM((EPW,), jnp.int32),
            pltpu.VMEM((HSZ,), jnp.float32),
        ],
        compiler_params=pltpu.CompilerParams(needs_layout_passes=False),
    )(_hist_body)
    return kern(dst)


def _elu(x):
    return jnp.where(x > 0, x, jnp.exp(jnp.minimum(x, 0.0)) - 1.0)


def _dot_t(a, w):
    return lax.dot_general(a, w, (((1,), (1,)), ((), ())),
                           preferred_element_type=jnp.float32)


def _gru_layer(hs, tw, tb, wih, bih, whh, bhh):
    cs = _elu(_dot_t(hs, tw) + tb)
    gi = _dot_t(cs, wih) + bih
    gh = _dot_t(hs, whh) + bhh
    r = jax.nn.sigmoid(gi[:, :D] + gh[:, :D])
    z = jax.nn.sigmoid(gi[:, D:2 * D] + gh[:, D:2 * D])
    n = jnp.tanh(gi[:, 2 * D:] + r * gh[:, 2 * D:])
    return (1.0 - z) * n + z * hs


def _dense_body(hist_ref, x_ref, tw0_ref, tb0_ref, wih0_ref, bih0_ref,
                whh0_ref, bhh0_ref, tw1_ref, tb1_ref, wih1_ref, bih1_ref,
                whh1_ref, bhh1_ref, out_ref):
    xv = x_ref[...]
    cnt = jnp.sum(hist_ref[...], axis=1, keepdims=True)
    s = jnp.sum(xv, axis=1, keepdims=True)
    hs0 = jnp.broadcast_to(s, xv.shape)
    h0 = _gru_layer(hs0, tw0_ref[...], tb0_ref[...], wih0_ref[...],
                    bih0_ref[...], whh0_ref[...], bhh0_ref[...])
    y = cnt * h0
    h1 = _gru_layer(y, tw1_ref[...], tb1_ref[...], wih1_ref[...],
                    bih1_ref[...], whh1_ref[...], bhh1_ref[...])
    out_ref[...] = cnt * h1


BLK = 1000


def _dense(hist, x, tw0, tb0, wih0, bih0, whh0, bhh0,
           tw1, tb1, wih1, bih1, whh1, bhh1):
    grid = (N // BLK,)
    full = lambda shape: pl.BlockSpec(shape, lambda i: (0, 0))
    return pl.pallas_call(
        _dense_body,
        grid=grid,
        in_specs=[
            pl.BlockSpec((BLK, NWORK * NSUB), lambda i: (i, 0)),
            pl.BlockSpec((BLK, D), lambda i: (i, 0)),
            full((D, D)), full((1, D)),
            full((3 * D, D)), full((1, 3 * D)),
            full((3 * D, D)), full((1, 3 * D)),
            full((D, D)), full((1, D)),
            full((3 * D, D)), full((1, 3 * D)),
            full((3 * D, D)), full((1, 3 * D)),
        ],
        out_specs=pl.BlockSpec((BLK, D), lambda i: (i, 0)),
        out_shape=jax.ShapeDtypeStruct((N, D), jnp.float32),
    )(hist, x, tw0, tb0, wih0, bih0, whh0, bhh0,
      tw1, tb1, wih1, bih1, whh1, bhh1)


def kernel(x, edge_index, align_w0, align_b0, attend_w0, attend_b0,
           gru_wih0, gru_whh0, gru_bih0, gru_bhh0, align_w1, align_b1,
           attend_w1, attend_b1, gru_wih1, gru_whh1, gru_bih1, gru_bhh1):
    hist = _in_degree(edge_index[1])
    hist = hist.reshape(NWORK, NPAD, NSUB).transpose(1, 0, 2)
    hist = hist.reshape(NPAD, NWORK * NSUB)
    return _dense(
        hist, x,
        attend_w0, attend_b0.reshape(1, D),
        gru_wih0, gru_bih0.reshape(1, 3 * D),
        gru_whh0, gru_bhh0.reshape(1, 3 * D),
        attend_w1, attend_b1.reshape(1, D),
        gru_wih1, gru_bih1.reshape(1, 3 * D),
        gru_whh1, gru_bhh1.reshape(1, 3 * D),
    )


# speedup vs baseline: 66.6851x; 1.3492x over previous
import functools

import jax
import jax.numpy as jnp
from jax import lax
from jax.experimental import pallas as pl
from jax.experimental.pallas import tpu as pltpu
from jax.experimental.pallas import tpu_sc as plsc

N = 10000
E = 320000
D = 128

NPAD = 10240
NWORK = 32
LANES = 16
NSUB = 2
HSZ = NPAD * NSUB
EPW = 9984
CHUNKC = 10496


NGRP = LANES // NSUB


def _hist_body(ei_hbm, out_hbm, chunk_v, hist_v):
    c_idx = lax.axis_index("c")
    s_idx = lax.axis_index("s")
    wid = s_idx * 2 + c_idx
    lane = lax.iota(jnp.int32, LANES)
    ones = jnp.full((LANES,), 1.0, jnp.float32)
    zeros = jnp.zeros((LANES,), jnp.float32)
    masks = [(lane >= NSUB * g) & (lane < NSUB * (g + 1)) for g in range(NGRP)]

    pltpu.sync_copy(ei_hbm.at[:, pl.ds(wid * EPW, CHUNKC)], chunk_v)

    def zero_body(i, carry):
        for j in range(8):
            hist_v[pl.ds((i * 8 + j) * LANES, LANES)] = zeros
        return carry

    lax.fori_loop(0, HSZ // (8 * LANES), zero_body, 0)

    def edge_body(i, carry):
        v = chunk_v[1, pl.ds(i * LANES, LANES)]
        t = v * NSUB + lane
        for g in range(NGRP):
            plsc.addupdate_scatter(hist_v, [t - NSUB * g], ones,
                                   mask=masks[g])
        return carry

    nv = jnp.where(wid == NWORK - 1, CHUNKC // LANES, EPW // LANES)
    lax.fori_loop(0, nv, edge_body, 0)

    pltpu.sync_copy(hist_v, out_hbm.at[wid])


def _in_degree(edge_index):
    mesh = plsc.VectorSubcoreMesh(core_axis_name="c", subcore_axis_name="s")
    kern = functools.partial(
        pl.kernel,
        mesh=mesh,
        out_type=jax.ShapeDtypeStruct((NWORK, HSZ), jnp.float32),
        scratch_types=[
            pltpu.VMEM((2, CHUNKC), jnp.int32),
            pltpu.VMEM((HSZ,), jnp.float32),
        ],
        compiler_params=pltpu.CompilerParams(needs_layout_passes=False),
    )(_hist_body)
    return kern(edge_index)


def _elu(x):
    return jnp.where(x > 0, x, jnp.exp(jnp.minimum(x, 0.0)) - 1.0)


def _dot_t(a, w):
    return lax.dot_general(a, w, (((1,), (1,)), ((), ())),
                           preferred_element_type=jnp.float32)


def _gru_layer(hs, tw, tb, wih, bih, whh, bhh):
    cs = _elu(_dot_t(hs, tw) + tb)
    gi = _dot_t(cs, wih) + bih
    gh = _dot_t(hs, whh) + bhh
    r = jax.nn.sigmoid(gi[:, :D] + gh[:, :D])
    z = jax.nn.sigmoid(gi[:, D:2 * D] + gh[:, D:2 * D])
    n = jnp.tanh(gi[:, 2 * D:] + r * gh[:, 2 * D:])
    return (1.0 - z) * n + z * hs


def _dense_body(hist_ref, x_ref, tw0_ref, tb0_ref, wih0_ref, bih0_ref,
                whh0_ref, bhh0_ref, tw1_ref, tb1_ref, wih1_ref, bih1_ref,
                whh1_ref, bhh1_ref, out_ref):
    xv = x_ref[...]
    cnt = jnp.sum(hist_ref[...], axis=1, keepdims=True)
    s = jnp.sum(xv, axis=1, keepdims=True)
    hs0 = jnp.broadcast_to(s, xv.shape)
    h0 = _gru_layer(hs0, tw0_ref[...], tb0_ref[...], wih0_ref[...],
                    bih0_ref[...], whh0_ref[...], bhh0_ref[...])
    y = cnt * h0
    h1 = _gru_layer(y, tw1_ref[...], tb1_ref[...], wih1_ref[...],
                    bih1_ref[...], whh1_ref[...], bhh1_ref[...])
    out_ref[...] = cnt * h1


BLK = 1000


def _dense(hist, x, tw0, tb0, wih0, bih0, whh0, bhh0,
           tw1, tb1, wih1, bih1, whh1, bhh1):
    grid = (N // BLK,)
    full = lambda shape: pl.BlockSpec(shape, lambda i: (0, 0))
    return pl.pallas_call(
        _dense_body,
        grid=grid,
        in_specs=[
            pl.BlockSpec((BLK, NWORK * NSUB), lambda i: (i, 0)),
            pl.BlockSpec((BLK, D), lambda i: (i, 0)),
            full((D, D)), full((1, D)),
            full((3 * D, D)), full((1, 3 * D)),
            full((3 * D, D)), full((1, 3 * D)),
            full((D, D)), full((1, D)),
            full((3 * D, D)), full((1, 3 * D)),
            full((3 * D, D)), full((1, 3 * D)),
        ],
        out_specs=pl.BlockSpec((BLK, D), lambda i: (i, 0)),
        out_shape=jax.ShapeDtypeStruct((N, D), jnp.float32),
    )(hist, x, tw0, tb0, wih0, bih0, whh0, bhh0,
      tw1, tb1, wih1, bih1, whh1, bhh1)


def kernel(x, edge_index, align_w0, align_b0, attend_w0, attend_b0,
           gru_wih0, gru_whh0, gru_bih0, gru_bhh0, align_w1, align_b1,
           attend_w1, attend_b1, gru_wih1, gru_whh1, gru_bih1, gru_bhh1):
    hist = _in_degree(edge_index)
    hist = hist.reshape(NWORK, NPAD, NSUB).transpose(1, 0, 2)
    hist = hist.reshape(NPAD, NWORK * NSUB)
    return _dense(
        hist, x,
        attend_w0, attend_b0.reshape(1, D),
        gru_wih0, gru_bih0.reshape(1, 3 * D),
        gru_whh0, gru_bhh0.reshape(1, 3 * D),
        attend_w1, attend_b1.reshape(1, D),
        gru_wih1, gru_bih1.reshape(1, 3 * D),
        gru_whh1, gru_bhh1.reshape(1, 3 * D),
    )


# speedup vs baseline: 98.3217x; 1.4744x over previous
import functools

import jax
import jax.numpy as jnp
from jax import lax
from jax.experimental import pallas as pl
from jax.experimental.pallas import tpu as pltpu
from jax.experimental.pallas import tpu_sc as plsc

N = 10000
E = 320000
D = 128

NPAD = 10240
NWORK = 32
LANES = 16
NSUB = 2
HSZ = NPAD * NSUB
EPW = 9984
CHUNKC = 10496


NGRP = LANES // NSUB


def _hist_body(ei_hbm, out_hbm, chunk_v, hist_v):
    c_idx = lax.axis_index("c")
    s_idx = lax.axis_index("s")
    wid = s_idx * 2 + c_idx
    lane = lax.iota(jnp.int32, LANES)
    ones = jnp.full((LANES,), 1.0, jnp.float32)
    zeros = jnp.zeros((LANES,), jnp.float32)
    masks = [(lane >= NSUB * g) & (lane < NSUB * (g + 1)) for g in range(NGRP)]

    pltpu.sync_copy(ei_hbm.at[:, pl.ds(wid * EPW, CHUNKC)], chunk_v)

    def zero_body(i, carry):
        for j in range(8):
            hist_v[pl.ds((i * 8 + j) * LANES, LANES)] = zeros
        return carry

    lax.fori_loop(0, HSZ // (8 * LANES), zero_body, 0)

    sub_off = (lane & 1) * NPAD

    def edge_body(i, carry):
        v = chunk_v[1, pl.ds(i * LANES, LANES)]
        t = sub_off + v
        for g in range(NGRP):
            plsc.addupdate_scatter(hist_v, [t], ones, mask=masks[g])
        return carry

    nv = jnp.where(wid == NWORK - 1, CHUNKC // LANES, EPW // LANES)
    lax.fori_loop(0, nv, edge_body, 0)

    pltpu.sync_copy(hist_v.at[pl.ds(0, NPAD)], out_hbm.at[NSUB * wid])
    pltpu.sync_copy(hist_v.at[pl.ds(NPAD, NPAD)], out_hbm.at[NSUB * wid + 1])


def _in_degree(edge_index):
    mesh = plsc.VectorSubcoreMesh(core_axis_name="c", subcore_axis_name="s")
    kern = functools.partial(
        pl.kernel,
        mesh=mesh,
        out_type=jax.ShapeDtypeStruct((NWORK * NSUB, NPAD), jnp.float32),
        scratch_types=[
            pltpu.VMEM((2, CHUNKC), jnp.int32),
            pltpu.VMEM((HSZ,), jnp.float32),
        ],
        compiler_params=pltpu.CompilerParams(needs_layout_passes=False),
    )(_hist_body)
    return kern(edge_index)


def _elu(x):
    return jnp.where(x > 0, x, jnp.exp(jnp.minimum(x, 0.0)) - 1.0)


def _dot_t(a, w):
    return lax.dot_general(a, w, (((1,), (1,)), ((), ())),
                           preferred_element_type=jnp.float32)


def _gru_layer(hs, tw, tb, wih, bih, whh, bhh):
    cs = _elu(_dot_t(hs, tw) + tb)
    gi = _dot_t(cs, wih) + bih
    gh = _dot_t(hs, whh) + bhh
    r = jax.nn.sigmoid(gi[:, :D] + gh[:, :D])
    z = jax.nn.sigmoid(gi[:, D:2 * D] + gh[:, D:2 * D])
    n = jnp.tanh(gi[:, 2 * D:] + r * gh[:, 2 * D:])
    return (1.0 - z) * n + z * hs


def _dense_body(hist_ref, x_ref, tw0_ref, tb0_ref, wih0_ref, bih0_ref,
                whh0_ref, bhh0_ref, tw1_ref, tb1_ref, wih1_ref, bih1_ref,
                whh1_ref, bhh1_ref, out_ref):
    xv = x_ref[...]
    hb = hist_ref[...]
    rr = lax.broadcasted_iota(jnp.int32, (NWORK * NSUB, NWORK * NSUB), 0)
    cc = lax.broadcasted_iota(jnp.int32, (NWORK * NSUB, NWORK * NSUB), 1)
    eye = (rr == cc).astype(jnp.float32)
    ht = lax.dot_general(hb, eye, (((0,), (0,)), ((), ())),
                         preferred_element_type=jnp.float32)
    cnt = jnp.sum(ht, axis=1, keepdims=True)
    s = jnp.sum(xv, axis=1, keepdims=True)
    hs0 = jnp.broadcast_to(s, xv.shape)
    h0 = _gru_layer(hs0, tw0_ref[...], tb0_ref[...], wih0_ref[...],
                    bih0_ref[...], whh0_ref[...], bhh0_ref[...])
    y = cnt * h0
    h1 = _gru_layer(y, tw1_ref[...], tb1_ref[...], wih1_ref[...],
                    bih1_ref[...], whh1_ref[...], bhh1_ref[...])
    out_ref[...] = cnt * h1


BLK = 1024


def _dense(hist, x, tw0, tb0, wih0, bih0, whh0, bhh0,
           tw1, tb1, wih1, bih1, whh1, bhh1):
    grid = (NPAD // BLK,)
    full = lambda shape: pl.BlockSpec(shape, lambda i: (0, 0))
    return pl.pallas_call(
        _dense_body,
        grid=grid,
        in_specs=[
            pl.BlockSpec((NWORK * NSUB, BLK), lambda i: (0, i)),
            pl.BlockSpec((BLK, D), lambda i: (i, 0)),
            full((D, D)), full((1, D)),
            full((3 * D, D)), full((1, 3 * D)),
            full((3 * D, D)), full((1, 3 * D)),
            full((D, D)), full((1, D)),
            full((3 * D, D)), full((1, 3 * D)),
            full((3 * D, D)), full((1, 3 * D)),
        ],
        out_specs=pl.BlockSpec((BLK, D), lambda i: (i, 0)),
        out_shape=jax.ShapeDtypeStruct((N, D), jnp.float32),
    )(hist, x, tw0, tb0, wih0, bih0, whh0, bhh0,
      tw1, tb1, wih1, bih1, whh1, bhh1)


def kernel(x, edge_index, align_w0, align_b0, attend_w0, attend_b0,
           gru_wih0, gru_whh0, gru_bih0, gru_bhh0, align_w1, align_b1,
           attend_w1, attend_b1, gru_wih1, gru_whh1, gru_bih1, gru_bhh1):
    hist = _in_degree(edge_index)
    return _dense(
        hist, x,
        attend_w0, attend_b0.reshape(1, D),
        gru_wih0, gru_bih0.reshape(1, 3 * D),
        gru_whh0, gru_bhh0.reshape(1, 3 * D),
        attend_w1, attend_b1.reshape(1, D),
        gru_wih1, gru_bih1.reshape(1, 3 * D),
        gru_whh1, gru_bhh1.reshape(1, 3 * D),
    )
